# Initial kernel scaffold; baseline (speedup 1.0000x reference)
import jax
import jax.numpy as jnp
from jax.experimental import pallas as pl


def kernel(x, pos_embedding):
    raise NotImplementedError("write your pallas kernel here")


# speedup vs baseline: 1.3832x; 1.3832x over previous
import jax
import jax.numpy as jnp
from jax import lax
from jax.experimental import pallas as pl
from jax.experimental.pallas import tpu as pltpu
from jax.experimental.pallas import tpu_sc as plsc

B, T, N, D = 16, 4, 1024, 256
IN_N = 512
NC, NS = 2, 16
NW = NC * NS
CHUNKS = NW // T
OUT_ROWS = N // CHUNKS
IN_ROWS = OUT_ROWS // 2
BUF_ROWS = IN_ROWS + 16
LANES = 16
VPR = D // LANES


def _body(pe_hbm, out_hbm, in_v, out_v, sem):
    wid = lax.axis_index("s") * NC + lax.axis_index("c")
    tt = wid // CHUNKS
    c = wid % CHUNKS
    k0 = c * IN_ROWS
    start = jnp.clip(k0 - 8, 0, IN_N - BUF_ROWS)
    src_row = pl.multiple_of(tt * IN_N + start, 8)
    pltpu.sync_copy(pe_hbm.at[pl.ds(src_row, BUF_ROWS)], in_v)

    w_lo = jnp.full((LANES,), 0.25, jnp.float32)
    w_hi = jnp.full((LANES,), 0.75, jnp.float32)

    def j_body(j, carry):
        row_a = jnp.maximum(k0 + j - 1, 0) - start
        row_b = k0 + j - start
        row_c = jnp.minimum(k0 + j + 1, IN_N - 1) - start
        for v in range(VPR):
            sl = pl.ds(v * LANES, LANES)
            a = in_v[row_a, sl]
            bb = in_v[row_b, sl]
            cc = in_v[row_c, sl]
            out_v[2 * j, sl] = w_lo * a + w_hi * bb
            out_v[2 * j + 1, sl] = w_hi * bb + w_lo * cc
        return carry

    lax.fori_loop(0, IN_ROWS, j_body, 0)

    i0 = c * OUT_ROWS
    copies = [
        pltpu.async_copy(
            out_v, out_hbm.at[pl.ds((b * T + tt) * N + i0, OUT_ROWS)], sem
        )
        for b in range(B)
    ]
    for cp in copies:
        cp.wait()


def kernel(x, pos_embedding):
    pe2 = pos_embedding.reshape(-1, D)
    mesh = plsc.VectorSubcoreMesh(core_axis_name="c", subcore_axis_name="s")
    f = pl.kernel(
        _body,
        out_type=jax.ShapeDtypeStruct((B * T * N, D), jnp.float32),
        mesh=mesh,
        scratch_types=[
            pltpu.VMEM((BUF_ROWS, D), jnp.float32),
            pltpu.VMEM((OUT_ROWS, D), jnp.float32),
            pltpu.SemaphoreType.DMA,
        ],
    )
    return f(pe2).reshape(B, T, N, D)


# speedup vs baseline: 1.4711x; 1.0636x over previous
import jax
import jax.numpy as jnp
from jax import lax
from jax.experimental import pallas as pl
from jax.experimental.pallas import tpu as pltpu
from jax.experimental.pallas import tpu_sc as plsc

B, T, N, D = 16, 4, 1024, 256
IN_N = 512
NC, NS = 2, 16
NW = NC * NS
CHUNKS = NW // T
OUT_ROWS = N // CHUNKS
IN_ROWS = OUT_ROWS // 2
BUF_ROWS = IN_ROWS + 16
LANES = 16
VPR = D // LANES
SUB = 4
SUB_J = IN_ROWS // SUB


def _body(pe_hbm, out_hbm, in_v, out_v, sem):
    wid = lax.axis_index("s") * NC + lax.axis_index("c")
    tt = wid // CHUNKS
    c = wid % CHUNKS
    k0 = c * IN_ROWS
    start = jnp.clip(k0 - 8, 0, IN_N - BUF_ROWS)
    src_row = pl.multiple_of(tt * IN_N + start, 8)
    pltpu.sync_copy(pe_hbm.at[pl.ds(src_row, BUF_ROWS)], in_v)

    w_lo = jnp.full((LANES,), 0.25, jnp.float32)
    w_hi = jnp.full((LANES,), 0.75, jnp.float32)

    def j_body(j, carry):
        row_a = jnp.maximum(k0 + j - 1, 0) - start
        row_b = k0 + j - start
        row_c = jnp.minimum(k0 + j + 1, IN_N - 1) - start
        for v in range(VPR):
            sl = pl.ds(v * LANES, LANES)
            a = in_v[row_a, sl]
            bb = in_v[row_b, sl]
            cc = in_v[row_c, sl]
            out_v[2 * j, sl] = w_lo * a + w_hi * bb
            out_v[2 * j + 1, sl] = w_hi * bb + w_lo * cc
        return carry

    i0 = c * OUT_ROWS
    copies = []
    for s in range(SUB):
        lax.fori_loop(s * SUB_J, (s + 1) * SUB_J, j_body, 0)
        src = out_v.at[pl.ds(s * 2 * SUB_J, 2 * SUB_J)]
        for b in range(B):
            dst_row = pl.multiple_of((b * T + tt) * N + i0 + s * 2 * SUB_J, 8)
            copies.append(
                pltpu.async_copy(src, out_hbm.at[pl.ds(dst_row, 2 * SUB_J)], sem)
            )
    for cp in copies:
        cp.wait()


def kernel(x, pos_embedding):
    pe2 = pos_embedding.reshape(-1, D)
    mesh = plsc.VectorSubcoreMesh(core_axis_name="c", subcore_axis_name="s")
    f = pl.kernel(
        _body,
        out_type=jax.ShapeDtypeStruct((B * T * N, D), jnp.float32),
        mesh=mesh,
        scratch_types=[
            pltpu.VMEM((BUF_ROWS, D), jnp.float32),
            pltpu.VMEM((OUT_ROWS, D), jnp.float32),
            pltpu.SemaphoreType.DMA,
        ],
    )
    return f(pe2).reshape(B, T, N, D)
